# baseline (speedup 1.0000x reference)
import jax
import jax.numpy as jnp
from jax import lax
from jax.experimental import pallas as pl
from jax.experimental.pallas import tpu as pltpu

_S3 = 8
_C = 128


def _feat_body(f_ref, o_ref, sem):
    i = pl.program_id(0)
    b = f_ref.shape[0]
    cps = [
        pltpu.make_async_copy(f_ref, o_ref.at[pl.ds(i * b, b), j], sem)
        for j in range(_S3)
    ]
    for c in cps:
        c.start()
    for c in cps:
        c.wait()


def _coord_body(c_ref, o_ref):
    b = c_ref.shape[1]
    rep = jnp.broadcast_to(c_ref[...][:, None, :], (3, _S3, b)).reshape(3 * _S3, b)
    rr = lax.broadcasted_iota(jnp.int32, (3 * _S3, b), 0)
    off = lax.shift_right_logical(rr % _S3, 2 - rr // _S3) & 1
    o_ref[...] = rep * 2 + off


def kernel(features, coords):
    n, c = features.shape
    bf = 10000
    fine3 = pl.pallas_call(
        _feat_body,
        grid=(n // bf,),
        in_specs=[pl.BlockSpec((bf, c), lambda i: (i, 0))],
        out_specs=pl.BlockSpec(memory_space=pl.ANY),
        out_shape=jax.ShapeDtypeStruct((n, _S3, c), jnp.float32),
        scratch_shapes=[pltpu.SemaphoreType.DMA],
    )(features)

    bc = 12544
    gc = -(-n // bc)
    fine_t2 = pl.pallas_call(
        _coord_body,
        grid=(gc,),
        in_specs=[pl.BlockSpec((3, bc), lambda i: (0, i))],
        out_specs=pl.BlockSpec((3 * _S3, bc), lambda i: (0, i)),
        out_shape=jax.ShapeDtypeStruct((3 * _S3, n), jnp.int32),
    )(coords.T)

    fine_coords = (
        fine_t2.reshape(3, _S3, n).transpose(2, 1, 0).reshape(n * _S3, 3)
    )
    return fine3.reshape(n * _S3, c), fine_coords


# speedup vs baseline: 1.6201x; 1.6201x over previous
import functools

import jax
import jax.numpy as jnp
from jax import lax
from jax.experimental import pallas as pl
from jax.experimental.pallas import tpu as pltpu
from jax.experimental.pallas import tpu_sc as plsc

_S3 = 8
_C = 128

_NW = 32
_PSTEP = 512
_IN_LEN = _PSTEP + 8


def _feat_body(f_ref, o_ref, sem):
    i = pl.program_id(0)
    b = f_ref.shape[0]
    cps = [
        pltpu.make_async_copy(f_ref, o_ref.at[pl.ds(i * b, b), j], sem)
        for j in range(_S3)
    ]
    for c in cps:
        c.start()
    for c in cps:
        c.wait()


def _coord_sc_body(c_hbm, o_hbm, cbuf, obuf):
    npar = c_hbm.shape[1]
    per_w = npar // _NW
    wid = lax.axis_index("s") * 2 + lax.axis_index("c")
    base = wid * per_w
    last = base + per_w - _PSTEP
    nsteps = -(-per_w // _PSTEP)

    lanes = lax.iota(jnp.int32, 16)
    pat = jnp.where(lanes < 8, 0, 1)
    offs = [((lanes & 7) >> (2 - k)) & 1 for k in range(3)]
    nvr = _S3 * _PSTEP // 16

    def step(s, _):
        p0 = jnp.minimum(base + s * _PSTEP, last)
        a0 = jnp.minimum((p0 // 8) * 8, npar - _IN_LEN)
        rel = p0 - a0
        for k in range(3):
            pltpu.sync_copy(
                c_hbm.at[k, pl.ds(a0, _IN_LEN)],
                cbuf.at[pl.ds(k * _IN_LEN, _IN_LEN)],
            )

        def vreg(v, _):
            q = 2 * v
            for k in range(3):
                idx = (k * _IN_LEN + q + rel) + pat
                val = plsc.load_gather(cbuf, [idx])
                obuf[k, pl.ds(v * 16, 16)] = val + val + offs[k]
            return 0

        lax.fori_loop(0, nvr, vreg, 0, unroll=2)
        t0 = _S3 * p0
        for k in range(3):
            pltpu.sync_copy(obuf.at[k], o_hbm.at[k, pl.ds(t0, _S3 * _PSTEP)])
        return 0

    lax.fori_loop(0, nsteps, step, 0)


def kernel(features, coords):
    n, c = features.shape
    bf = 10000
    fine3 = pl.pallas_call(
        _feat_body,
        grid=(n // bf,),
        in_specs=[pl.BlockSpec((bf, c), lambda i: (i, 0))],
        out_specs=pl.BlockSpec(memory_space=pl.ANY),
        out_shape=jax.ShapeDtypeStruct((n, _S3, c), jnp.float32),
        scratch_shapes=[pltpu.SemaphoreType.DMA],
    )(features)

    sc_coords = functools.partial(
        pl.kernel,
        out_type=jax.ShapeDtypeStruct((3, n * _S3), jnp.int32),
        mesh=plsc.VectorSubcoreMesh(core_axis_name="c", subcore_axis_name="s"),
        compiler_params=pltpu.CompilerParams(
            use_tc_tiling_on_sc=False, needs_layout_passes=False
        ),
        scratch_types=[
            pltpu.VMEM((3 * _IN_LEN,), jnp.int32),
            pltpu.VMEM((3, _S3 * _PSTEP), jnp.int32),
        ],
    )(_coord_sc_body)
    fine_t = sc_coords(coords.T)

    return fine3.reshape(n * _S3, c), fine_t.T


# speedup vs baseline: 1.6467x; 1.0164x over previous
import functools

import jax
import jax.numpy as jnp
from jax import lax
from jax.experimental import pallas as pl
from jax.experimental.pallas import tpu as pltpu
from jax.experimental.pallas import tpu_sc as plsc

_S3 = 8
_C = 128

_NW = 32
_PSTEP = 512
_IN_LEN = _PSTEP + 8


def _feat_body(f_ref, o_ref, sem):
    i = pl.program_id(0)
    b = f_ref.shape[0]
    cps = [
        pltpu.make_async_copy(f_ref, o_ref.at[pl.ds(i * b, b), j], sem)
        for j in range(_S3)
    ]
    for c in cps:
        c.start()
    for c in cps:
        c.wait()


def _coord_sc_body(c_hbm, o_hbm, cbuf, obuf):
    npar = c_hbm.shape[1]
    per_w = npar // _NW
    wid = lax.axis_index("s") * 2 + lax.axis_index("c")
    base = wid * per_w
    last = base + per_w - _PSTEP
    nsteps = -(-per_w // _PSTEP)

    lanes = lax.iota(jnp.int32, 16)
    pat = jnp.where(lanes < 8, 0, 1)
    offs = [((lanes & 7) >> (2 - k)) & 1 for k in range(3)]
    nvr = _S3 * _PSTEP // 16

    def step(s, _):
        p0 = jnp.minimum(base + s * _PSTEP, last)
        a0 = jnp.minimum((p0 // 8) * 8, npar - _IN_LEN)
        rel = p0 - a0
        for k in range(3):
            pltpu.sync_copy(
                c_hbm.at[k, pl.ds(a0, _IN_LEN)],
                cbuf.at[pl.ds(k * _IN_LEN, _IN_LEN)],
            )

        def vreg(v, _):
            q = 2 * v
            for k in range(3):
                idx = (k * _IN_LEN + q + rel) + pat
                val = plsc.load_gather(cbuf, [idx])
                obuf[k, pl.ds(v * 16, 16)] = val + val + offs[k]
            return 0

        lax.fori_loop(0, nvr, vreg, 0, unroll=2)
        t0 = _S3 * p0
        for k in range(3):
            pltpu.sync_copy(obuf.at[k], o_hbm.at[k, pl.ds(t0, _S3 * _PSTEP)])
        return 0

    lax.fori_loop(0, nsteps, step, 0)


def kernel(features, coords):
    n, c = features.shape
    bf = 20000
    fine3 = pl.pallas_call(
        _feat_body,
        grid=(n // bf,),
        in_specs=[pl.BlockSpec((bf, c), lambda i: (i, 0))],
        out_specs=pl.BlockSpec(memory_space=pl.ANY),
        out_shape=jax.ShapeDtypeStruct((n, _S3, c), jnp.float32),
        scratch_shapes=[pltpu.SemaphoreType.DMA],
    )(features)

    sc_coords = functools.partial(
        pl.kernel,
        out_type=jax.ShapeDtypeStruct((3, n * _S3), jnp.int32),
        mesh=plsc.VectorSubcoreMesh(core_axis_name="c", subcore_axis_name="s"),
        compiler_params=pltpu.CompilerParams(
            use_tc_tiling_on_sc=False, needs_layout_passes=False
        ),
        scratch_types=[
            pltpu.VMEM((3 * _IN_LEN,), jnp.int32),
            pltpu.VMEM((3, _S3 * _PSTEP), jnp.int32),
        ],
    )(_coord_sc_body)
    fine_t = sc_coords(coords.T)

    return fine3.reshape(n * _S3, c), fine_t.T


# speedup vs baseline: 1.6641x; 1.0106x over previous
import functools

import jax
import jax.numpy as jnp
from jax import lax
from jax.experimental import pallas as pl
from jax.experimental.pallas import tpu as pltpu
from jax.experimental.pallas import tpu_sc as plsc

_S3 = 8
_C = 128

_NW = 32
_PSTEP = 512
_IN_LEN = _PSTEP + 8


def _feat_body(f_ref, o_ref, sem):
    i = pl.program_id(0)
    b = f_ref.shape[0]
    cps = [
        pltpu.make_async_copy(f_ref, o_ref.at[pl.ds(i * b, b), j], sem)
        for j in range(_S3)
    ]
    for c in cps:
        c.start()
    for c in cps:
        c.wait()


def _coord_sc_body(c_hbm, o_hbm, cbuf, obuf):
    npar = c_hbm.shape[1]
    per_w = npar // _NW
    wid = lax.axis_index("s") * 2 + lax.axis_index("c")
    base = wid * per_w
    last = base + per_w - _PSTEP
    nsteps = -(-per_w // _PSTEP)

    lanes = lax.iota(jnp.int32, 16)
    pat = jnp.where(lanes < 8, 0, 1)
    offs = [((lanes & 7) >> (2 - k)) & 1 for k in range(3)]
    nvr = _S3 * _PSTEP // 16

    def step(s, _):
        p0 = jnp.minimum(base + s * _PSTEP, last)
        a0 = jnp.minimum((p0 // 8) * 8, npar - _IN_LEN)
        rel = p0 - a0
        for k in range(3):
            pltpu.sync_copy(
                c_hbm.at[k, pl.ds(a0, _IN_LEN)],
                cbuf.at[pl.ds(k * _IN_LEN, _IN_LEN)],
            )

        def vreg(v, _):
            q = 2 * v
            for k in range(3):
                idx = (k * _IN_LEN + q + rel) + pat
                val = plsc.load_gather(cbuf, [idx])
                obuf[k, pl.ds(v * 16, 16)] = val + val + offs[k]
            return 0

        lax.fori_loop(0, nvr, vreg, 0, unroll=2)
        t0 = _S3 * p0
        for k in range(3):
            pltpu.sync_copy(obuf.at[k], o_hbm.at[k, pl.ds(t0, _S3 * _PSTEP)])
        return 0

    lax.fori_loop(0, nsteps, step, 0)


def kernel(features, coords):
    n, c = features.shape
    bf = 25000
    fine3 = pl.pallas_call(
        _feat_body,
        grid=(n // bf,),
        in_specs=[pl.BlockSpec((bf, c), lambda i: (i, 0))],
        out_specs=pl.BlockSpec(memory_space=pl.ANY),
        out_shape=jax.ShapeDtypeStruct((n, _S3, c), jnp.float32),
        scratch_shapes=[pltpu.SemaphoreType.DMA],
    )(features)

    sc_coords = functools.partial(
        pl.kernel,
        out_type=jax.ShapeDtypeStruct((3, n * _S3), jnp.int32),
        mesh=plsc.VectorSubcoreMesh(core_axis_name="c", subcore_axis_name="s"),
        compiler_params=pltpu.CompilerParams(
            use_tc_tiling_on_sc=False, needs_layout_passes=False
        ),
        scratch_types=[
            pltpu.VMEM((3 * _IN_LEN,), jnp.int32),
            pltpu.VMEM((3, _S3 * _PSTEP), jnp.int32),
        ],
    )(_coord_sc_body)
    fine_t = sc_coords(coords.T)

    return fine3.reshape(n * _S3, c), fine_t.T


# speedup vs baseline: 1.6662x; 1.0013x over previous
import functools

import jax
import jax.numpy as jnp
from jax import lax
from jax.experimental import pallas as pl
from jax.experimental.pallas import tpu as pltpu
from jax.experimental.pallas import tpu_sc as plsc

_S3 = 8
_C = 128

_NW = 32
_PSTEP = 512
_IN_LEN = _PSTEP + 8


def _feat_body(f_ref, o_ref, sem):
    i = pl.program_id(0)
    b = f_ref.shape[0]
    cps = [
        pltpu.make_async_copy(f_ref, o_ref.at[pl.ds(i * b, b), j], sem)
        for j in range(_S3)
    ]
    for c in cps:
        c.start()
    for c in cps:
        c.wait()


def _coord_sc_body(c_hbm, o_hbm, cbuf, obuf):
    npar = c_hbm.shape[1]
    per_w = npar // _NW
    wid = lax.axis_index("s") * 2 + lax.axis_index("c")
    base = wid * per_w
    last = base + per_w - _PSTEP
    nsteps = -(-per_w // _PSTEP)

    lanes = lax.iota(jnp.int32, 16)
    pat = jnp.where(lanes < 8, 0, 1)
    offs = [((lanes & 7) >> (2 - k)) & 1 for k in range(3)]
    nvr = _S3 * _PSTEP // 16

    def step(s, _):
        p0 = jnp.minimum(base + s * _PSTEP, last)
        a0 = jnp.minimum((p0 // 8) * 8, npar - _IN_LEN)
        rel = p0 - a0
        for k in range(3):
            pltpu.sync_copy(
                c_hbm.at[k, pl.ds(a0, _IN_LEN)],
                cbuf.at[pl.ds(k * _IN_LEN, _IN_LEN)],
            )

        def vreg(v, _):
            q = 2 * v
            for k in range(3):
                idx = (k * _IN_LEN + q + rel) + pat
                val = plsc.load_gather(cbuf, [idx])
                obuf[k, pl.ds(v * 16, 16)] = val + val + offs[k]
            return 0

        lax.fori_loop(0, nvr, vreg, 0, unroll=2)
        t0 = _S3 * p0
        for k in range(3):
            pltpu.sync_copy(obuf.at[k], o_hbm.at[k, pl.ds(t0, _S3 * _PSTEP)])
        return 0

    lax.fori_loop(0, nsteps, step, 0)


def kernel(features, coords):
    n, c = features.shape
    bf = 50000
    fine3 = pl.pallas_call(
        _feat_body,
        grid=(n // bf,),
        in_specs=[pl.BlockSpec((bf, c), lambda i: (i, 0))],
        out_specs=pl.BlockSpec(memory_space=pl.ANY),
        out_shape=jax.ShapeDtypeStruct((n, _S3, c), jnp.float32),
        scratch_shapes=[pltpu.SemaphoreType.DMA],
    )(features)

    sc_coords = functools.partial(
        pl.kernel,
        out_type=jax.ShapeDtypeStruct((3, n * _S3), jnp.int32),
        mesh=plsc.VectorSubcoreMesh(core_axis_name="c", subcore_axis_name="s"),
        compiler_params=pltpu.CompilerParams(
            use_tc_tiling_on_sc=False, needs_layout_passes=False
        ),
        scratch_types=[
            pltpu.VMEM((3 * _IN_LEN,), jnp.int32),
            pltpu.VMEM((3, _S3 * _PSTEP), jnp.int32),
        ],
    )(_coord_sc_body)
    fine_t = sc_coords(coords.T)

    return fine3.reshape(n * _S3, c), fine_t.T
